# baseline (speedup 1.0000x reference)
import functools

import jax
import jax.numpy as jnp
from jax import lax
from jax.experimental import pallas as pl
from jax.experimental.pallas import tpu as pltpu
from jax.experimental.pallas import tpu_sc as plsc

NC = 2
NS = 16
NW = NC * NS

CHUNK = 512
NBUF = 2
DEPTH = 1


def _make_gather(total, d):
    assert total % (NW * CHUNK) == 0
    per_w = total // NW
    n_chunks = per_w // CHUNK
    assert n_chunks % NBUF == 0
    mesh = plsc.VectorSubcoreMesh(core_axis_name="c", subcore_axis_name="s")

    @functools.partial(
        pl.kernel,
        mesh=mesh,
        out_type=jax.ShapeDtypeStruct((total, d), jnp.float32),
        compiler_params=pltpu.CompilerParams(use_tc_tiling_on_sc=False),
        scratch_types=[
            pltpu.VMEM((n_chunks, CHUNK), jnp.int32),
            pltpu.VMEM((NBUF, CHUNK, d), jnp.float32),
            pltpu.SemaphoreType.DMA((NBUF,)),
            pltpu.SemaphoreType.DMA((NBUF,)),
        ],
    )
    def gather_kernel(table_hbm, idx_hbm, out_hbm, idx_v, rows_v, gsem, osem):
        wid = lax.axis_index("s") * NC + lax.axis_index("c")
        row0 = wid * per_w

        pltpu.sync_copy(idx_hbm.at[wid], idx_v)

        def start_gather(j, b):
            pltpu.make_async_copy(
                table_hbm.at[idx_v.at[j]], rows_v.at[b], gsem.at[b]
            ).start()

        def wait_gather(j, b):
            pltpu.make_async_copy(
                table_hbm.at[idx_v.at[j]], rows_v.at[b], gsem.at[b]
            ).wait()

        def start_out(j, b):
            pltpu.make_async_copy(
                rows_v.at[b], out_hbm.at[pl.ds(row0 + j * CHUNK, CHUNK)],
                osem.at[b],
            ).start()

        def wait_out(j, b):
            pltpu.make_async_copy(
                rows_v.at[b], out_hbm.at[pl.ds(row0 + j * CHUNK, CHUNK)],
                osem.at[b],
            ).wait()

        for g in range(DEPTH):
            start_gather(g, g)

        def outer(i, carry):
            j0 = i * NBUF
            for b in range(NBUF):
                j = j0 + b
                wait_gather(j, b)
                start_out(j, b)
                g = j + DEPTH
                bg = (b + DEPTH) % NBUF

                @pl.when(jnp.logical_and(g < n_chunks, g >= NBUF))
                def _():
                    wait_out(g - NBUF, bg)

                @pl.when(g < n_chunks)
                def _():
                    start_gather(g, bg)
            return carry

        lax.fori_loop(0, n_chunks // NBUF, outer, 0)

        for b in range(NBUF):
            wait_out(n_chunks - NBUF + b, b)

    return gather_kernel


@jax.jit
def kernel(x, weight):
    batch, n_fields = x.shape
    total = batch * n_fields
    d = weight.shape[1]
    per_w = total // NW
    idx = x.reshape(NW, per_w // CHUNK, CHUNK).astype(jnp.int32)
    out = _make_gather(total, d)(weight, idx)
    return out.reshape(batch, n_fields, d)


# speedup vs baseline: 1.0773x; 1.0773x over previous
import functools

import jax
import jax.numpy as jnp
from jax import lax
from jax.experimental import pallas as pl
from jax.experimental.pallas import tpu as pltpu
from jax.experimental.pallas import tpu_sc as plsc

NC = 2
NS = 16
NW = NC * NS

CHUNK = 128
NBUF = 8
DEPTH = 4


def _make_gather(total, d):
    assert total % (NW * CHUNK) == 0
    per_w = total // NW
    n_chunks = per_w // CHUNK
    assert n_chunks % NBUF == 0
    mesh = plsc.VectorSubcoreMesh(core_axis_name="c", subcore_axis_name="s")

    @functools.partial(
        pl.kernel,
        mesh=mesh,
        out_type=jax.ShapeDtypeStruct((total, d), jnp.float32),
        compiler_params=pltpu.CompilerParams(use_tc_tiling_on_sc=False),
        scratch_types=[
            pltpu.VMEM((n_chunks, CHUNK), jnp.int32),
            pltpu.VMEM((NBUF, CHUNK, d), jnp.float32),
            pltpu.SemaphoreType.DMA((NBUF,)),
            pltpu.SemaphoreType.DMA((NBUF,)),
        ],
    )
    def gather_kernel(table_hbm, idx_hbm, out_hbm, idx_v, rows_v, gsem, osem):
        wid = lax.axis_index("s") * NC + lax.axis_index("c")
        row0 = wid * per_w

        pltpu.sync_copy(idx_hbm.at[wid], idx_v)

        def start_gather(j, b):
            pltpu.make_async_copy(
                table_hbm.at[idx_v.at[j]], rows_v.at[b], gsem.at[b]
            ).start()

        def wait_gather(j, b):
            pltpu.make_async_copy(
                table_hbm.at[idx_v.at[j]], rows_v.at[b], gsem.at[b]
            ).wait()

        def start_out(j, b):
            pltpu.make_async_copy(
                rows_v.at[b], out_hbm.at[pl.ds(row0 + j * CHUNK, CHUNK)],
                osem.at[b],
            ).start()

        def wait_out(j, b):
            pltpu.make_async_copy(
                rows_v.at[b], out_hbm.at[pl.ds(row0 + j * CHUNK, CHUNK)],
                osem.at[b],
            ).wait()

        for g in range(DEPTH):
            start_gather(g, g)

        def outer(i, carry):
            j0 = i * NBUF
            for b in range(NBUF):
                j = j0 + b
                wait_gather(j, b)
                start_out(j, b)
                g = j + DEPTH
                bg = (b + DEPTH) % NBUF

                @pl.when(jnp.logical_and(g < n_chunks, g >= NBUF))
                def _():
                    wait_out(g - NBUF, bg)

                @pl.when(g < n_chunks)
                def _():
                    start_gather(g, bg)
            return carry

        lax.fori_loop(0, n_chunks // NBUF, outer, 0)

        for b in range(NBUF):
            wait_out(n_chunks - NBUF + b, b)

    return gather_kernel


@jax.jit
def kernel(x, weight):
    batch, n_fields = x.shape
    total = batch * n_fields
    d = weight.shape[1]
    per_w = total // NW
    idx = (x.reshape(NW, per_w // CHUNK, CHUNK).astype(jnp.int32)) * 2
    wp = jnp.pad(weight, ((0, 0), (0, d))).reshape(2 * weight.shape[0], d)
    out = _make_gather(total, d)(wp, idx)
    return out.reshape(batch, n_fields, d)


# speedup vs baseline: 1.3417x; 1.2454x over previous
import functools

import jax
import jax.numpy as jnp
from jax import lax
from jax.experimental import pallas as pl
from jax.experimental.pallas import tpu as pltpu
from jax.experimental.pallas import tpu_sc as plsc

NC = 2
NS = 16
NW = NC * NS

BPC = 16
NBUF = 4
DEPTH = 2

FPAD = 32
DPAD = 128


def _make_gather(batch, n_fields, d):
    per_b = batch // NW
    chunk = BPC * n_fields
    n_chunks = per_b // BPC
    assert batch % (NW * BPC) == 0 and n_chunks % NBUF == 0
    mesh = plsc.VectorSubcoreMesh(core_axis_name="c", subcore_axis_name="s")

    @functools.partial(
        pl.kernel,
        mesh=mesh,
        out_type=jax.ShapeDtypeStruct((batch, FPAD, DPAD), jnp.float32),
        compiler_params=pltpu.CompilerParams(use_tc_tiling_on_sc=False),
        scratch_types=[
            pltpu.VMEM((n_chunks, chunk), jnp.int32),
            pltpu.VMEM((NBUF, chunk, d), jnp.float32),
            pltpu.SemaphoreType.DMA((NBUF,)),
            pltpu.SemaphoreType.DMA((NBUF,)),
        ],
    )
    def gather_kernel(table_hbm, idx_hbm, out_hbm, idx_v, rows_v, gsem, osem):
        wid = lax.axis_index("s") * NC + lax.axis_index("c")
        b0 = wid * per_b

        pltpu.sync_copy(idx_hbm.at[wid], idx_v)

        def start_gather(j, b):
            pltpu.make_async_copy(
                table_hbm.at[idx_v.at[j]], rows_v.at[b], gsem.at[b]
            ).start()

        def wait_gather(j, b):
            pltpu.make_async_copy(
                table_hbm.at[idx_v.at[j]], rows_v.at[b], gsem.at[b]
            ).wait()

        def out_copies(j, b):
            for k in range(BPC):
                yield pltpu.make_async_copy(
                    rows_v.at[b, pl.ds(k * n_fields, n_fields)],
                    out_hbm.at[b0 + j * BPC + k, pl.ds(0, n_fields),
                               pl.ds(0, d)],
                    osem.at[b],
                )

        def start_out(j, b):
            for c in out_copies(j, b):
                c.start()

        def wait_out(j, b):
            for c in out_copies(j, b):
                c.wait()

        for g in range(DEPTH):
            start_gather(g, g)

        def outer(i, carry):
            j0 = i * NBUF
            for b in range(NBUF):
                j = j0 + b
                wait_gather(j, b)
                start_out(j, b)
                g = j + DEPTH
                bg = (b + DEPTH) % NBUF

                @pl.when(jnp.logical_and(g < n_chunks, g >= NBUF))
                def _():
                    wait_out(g - NBUF, bg)

                @pl.when(g < n_chunks)
                def _():
                    start_gather(g, bg)
            return carry

        lax.fori_loop(0, n_chunks // NBUF, outer, 0)

        for b in range(NBUF):
            wait_out(n_chunks - NBUF + b, b)

    return gather_kernel


@jax.jit
def kernel(x, weight):
    batch, n_fields = x.shape
    d = weight.shape[1]
    per_b = batch // NW
    idx = (
        x.reshape(NW, per_b // BPC, BPC * n_fields).astype(jnp.int32) * 2
    )
    wp = jnp.pad(weight, ((0, 0), (0, d))).reshape(2 * weight.shape[0], d)
    out = _make_gather(batch, n_fields, d)(wp, idx)
    return out[:, :n_fields, :d]
